# baseline (speedup 1.0000x reference)
import functools

import jax
import jax.numpy as jnp
from jax import lax
from jax.experimental import pallas as pl
from jax.experimental.pallas import tpu as pltpu
from jax.experimental.pallas import tpu_sc as plsc

N = 10000
E = 320000
H = 128
NRG = 3333
ERG = 6666
L = 2

NC = 2
NS = 16
NW = NC * NS

ENC_CH = 3
NPAD_ENC = NW * ENC_CH * 128
NRG_PAD = 3584
RG_PER_SUB = NRG_PAD // NS
ECH = 80
EGRP = 8
EPW = ECH * 128
EPAD = NW * EPW
NAGG = 10112
AGG_PER_SUB = NAGG // NS
RECH = 2
REPW = RECH * 128
REPAD = NW * REPW

_mesh = plsc.VectorSubcoreMesh(core_axis_name="c", subcore_axis_name="s")


def _wid():
    return lax.axis_index("c") * NS + lax.axis_index("s")


def _zero_buf(buf, rows):
    z = jnp.zeros((16,), jnp.float32)

    @pl.loop(0, rows)
    def _(r):
        for k in range(H // 16):
            buf[r, pl.ds(16 * k, 16)] = z


def _relu_buf(buf, rows):
    @pl.loop(0, rows)
    def _(r):
        for k in range(H // 16):
            v = buf[r, pl.ds(16 * k, 16)]
            buf[r, pl.ds(16 * k, 16)] = jnp.maximum(v, 0.0)


def _encode_body(atab_hbm, rgtab_hbm, aidx_hbm, ridx_hbm,
                 x0_hbm, rg0_hbm,
                 atab_sh, rgtab_sh, aidx_v, ridx_v, buf, sem):
    cid = lax.axis_index("c")
    sid = lax.axis_index("s")
    w = cid * NS + sid

    @pl.when(sid == 0)
    def _():
        pltpu.sync_copy(atab_hbm, atab_sh)
        pltpu.sync_copy(rgtab_hbm, rgtab_sh)

    plsc.subcore_barrier()

    for f in range(9):
        pltpu.sync_copy(aidx_hbm.at[f, w], aidx_v.at[f])
    pltpu.sync_copy(ridx_hbm.at[w], ridx_v)

    @pl.loop(0, ENC_CH)
    def _(c):
        pltpu.async_copy(atab_sh.at[aidx_v.at[0, c]], buf, sem).wait()
        for f in range(1, 9):
            pltpu.async_copy(atab_sh.at[aidx_v.at[f, c]], buf, sem,
                             add=True).wait()
        pltpu.sync_copy(buf, x0_hbm.at[pl.ds((w * ENC_CH + c) * 128, 128)])

    rbuf = buf.at[pl.ds(0, 112)]
    pltpu.async_copy(rgtab_sh.at[ridx_v], rbuf, sem).wait()
    pltpu.sync_copy(rbuf, rg0_hbm.at[pl.ds(w * 112, 112)])


def _sc_encode(atab, rgtab, aidx, ridx):
    k = pl.kernel(
        _encode_body,
        out_type=[jax.ShapeDtypeStruct((NPAD_ENC, H), jnp.float32),
                  jax.ShapeDtypeStruct((NRG_PAD, H), jnp.float32)],
        mesh=_mesh,
        scratch_types=[
            pltpu.VMEM_SHARED((900, H), jnp.float32),
            pltpu.VMEM_SHARED((8, H), jnp.float32),
            pltpu.VMEM((9, ENC_CH, 128), jnp.int32),
            pltpu.VMEM((112,), jnp.int32),
            pltpu.VMEM((128, H), jnp.float32),
            pltpu.SemaphoreType.DMA,
        ],
    )
    return k(atab, rgtab, aidx, ridx)


def _edge_body(x_hbm, t01_hbm, t2_hbm, src_hbm, c01_hbm, c2_hbm, dst_hbm,
               agg_hbm,
               t2_sh, agg_sh,
               srcv, c01v, c2v, dstv, buf0, buf1, sem0, sem1, isem):
    cid = lax.axis_index("c")
    sid = lax.axis_index("s")
    w = cid * NS + sid

    @pl.when(sid == 0)
    def _():
        pltpu.sync_copy(t2_hbm, t2_sh)

    _zero_buf(buf0, 128)
    for k in range(AGG_PER_SUB // 128):
        pltpu.sync_copy(buf0, agg_sh.at[pl.ds(sid * AGG_PER_SUB + 128 * k, 128)])
    rem = AGG_PER_SUB % 128
    if rem:
        pltpu.sync_copy(
            buf0.at[pl.ds(0, rem)],
            agg_sh.at[pl.ds(sid * AGG_PER_SUB + AGG_PER_SUB - rem, rem)])
    plsc.subcore_barrier()

    bufs = (buf0, buf1)
    sems = (sem0, sem1)

    @pl.loop(0, ECH // EGRP)
    def _(g):
        ds = [pltpu.async_copy(src_hbm.at[w, pl.ds(g * EGRP, EGRP)], srcv, isem),
              pltpu.async_copy(dst_hbm.at[w, pl.ds(g * EGRP, EGRP)], dstv, isem),
              pltpu.async_copy(c01_hbm.at[w, pl.ds(g * EGRP, EGRP)], c01v, isem),
              pltpu.async_copy(c2_hbm.at[w, pl.ds(g * EGRP, EGRP)], c2v, isem)]
        for d in ds:
            d.wait()
        pltpu.async_copy(t01_hbm.at[c01v.at[0]], bufs[0], sems[0]).wait()
        for c in range(EGRP):
            cur = c % 2
            nxt = 1 - cur
            pltpu.async_copy(t2_sh.at[c2v.at[c]], bufs[cur], sems[cur],
                             add=True).wait()
            pltpu.async_copy(x_hbm.at[srcv.at[c]], bufs[cur], sems[cur],
                             add=True).wait()
            if c + 1 < EGRP:
                t01n = pltpu.async_copy(t01_hbm.at[c01v.at[c + 1]],
                                        bufs[nxt], sems[nxt])
            _relu_buf(bufs[cur], 128)
            pltpu.sync_copy(bufs[cur], agg_sh.at[dstv.at[c]], add=True)
            if c + 1 < EGRP:
                t01n.wait()

    plsc.subcore_barrier()

    pltpu.sync_copy(agg_sh.at[pl.ds(sid * AGG_PER_SUB, AGG_PER_SUB)],
                    agg_hbm.at[cid, pl.ds(sid * AGG_PER_SUB, AGG_PER_SUB)])


def _sc_edge(x, t01, t2, srcp, c01p, c2p, dstp):
    k = pl.kernel(
        _edge_body,
        out_type=[jax.ShapeDtypeStruct((NC, NAGG, H), jnp.float32)],
        mesh=_mesh,
        scratch_types=[
            pltpu.VMEM_SHARED((100, H), jnp.float32),
            pltpu.VMEM_SHARED((NAGG, H), jnp.float32),
            pltpu.VMEM((EGRP, 128), jnp.int32),
            pltpu.VMEM((EGRP, 128), jnp.int32),
            pltpu.VMEM((EGRP, 128), jnp.int32),
            pltpu.VMEM((EGRP, 128), jnp.int32),
            pltpu.VMEM((128, H), jnp.float32),
            pltpu.VMEM((128, H), jnp.float32),
            pltpu.SemaphoreType.DMA,
            pltpu.SemaphoreType.DMA,
            pltpu.SemaphoreType.DMA,
        ],
    )
    (agg,) = k(x, t01, t2, srcp, c01p, c2p, dstp)
    return agg


def _pair_body(b0_ref, b1_ref, t_ref):
    b0 = b0_ref[...]
    b1 = b1_ref[...]
    t_ref[...] = (b0[:, None, :] + b1[None, :, :]).reshape(10000, H)


def _pair_table(b0, b1):
    return pl.pallas_call(
        _pair_body,
        out_shape=jax.ShapeDtypeStruct((10000, H), jnp.float32),
    )(b0, b1)


def _rg_body(rg_hbm, rsrc_hbm, rdst_hbm, ragg_hbm,
             ragg_sh, rsv, rdv, buf, sem):
    cid = lax.axis_index("c")
    sid = lax.axis_index("s")
    w = cid * NS + sid

    _zero_buf(buf, 128)
    pltpu.sync_copy(buf, ragg_sh.at[pl.ds(sid * RG_PER_SUB, 128)])
    pltpu.sync_copy(buf.at[pl.ds(0, RG_PER_SUB - 128)],
                    ragg_sh.at[pl.ds(sid * RG_PER_SUB + 128, RG_PER_SUB - 128)])
    plsc.subcore_barrier()

    pltpu.sync_copy(rsrc_hbm.at[w], rsv)
    pltpu.sync_copy(rdst_hbm.at[w], rdv)

    for c in range(RECH):
        pltpu.async_copy(rg_hbm.at[rsv.at[c]], buf, sem).wait()
        pltpu.sync_copy(buf, ragg_sh.at[rdv.at[c]], add=True)

    plsc.subcore_barrier()
    pltpu.sync_copy(ragg_sh.at[pl.ds(sid * RG_PER_SUB, RG_PER_SUB)],
                    ragg_hbm.at[cid, pl.ds(sid * RG_PER_SUB, RG_PER_SUB)])


def _sc_rg(rg, rsrcp, rdstp):
    k = pl.kernel(
        _rg_body,
        out_type=[jax.ShapeDtypeStruct((NC, NRG_PAD, H), jnp.float32)],
        mesh=_mesh,
        scratch_types=[
            pltpu.VMEM_SHARED((NRG_PAD, H), jnp.float32),
            pltpu.VMEM((RECH, 128), jnp.int32),
            pltpu.VMEM((RECH, 128), jnp.int32),
            pltpu.VMEM((128, H), jnp.float32),
            pltpu.SemaphoreType.DMA,
        ],
    )
    (ragg,) = k(rg, rsrcp, rdstp)
    return ragg


def _bf16_dot(a, b):
    return jnp.dot(a.astype(jnp.bfloat16), b.astype(jnp.bfloat16),
                   preferred_element_type=jnp.float32)


def _acc_stats(stats_ref, z, valid_rows, first):
    @pl.when(first)
    def _():
        stats_ref[...] = jnp.zeros_like(stats_ref)

    if valid_rows is not None:
        mask = lax.broadcasted_iota(jnp.int32, z.shape, 0) < valid_rows
        z = jnp.where(mask, z, 0.0)
    stats_ref[0:1] += jnp.sum(z, axis=0, keepdims=True)
    stats_ref[1:2] += jnp.sum(z * z, axis=0, keepdims=True)


def _passA_body(valid_rows, x_ref, a0_ref, a1_ref, w_ref, z_ref, stats_ref):
    i = pl.program_id(0)
    h = x_ref[...] + (a0_ref[0] + a1_ref[0])
    z = _bf16_dot(h, w_ref[...])
    z_ref[...] = z
    _acc_stats(stats_ref, z, valid_rows, i == 0)


def _passB_body(cnt, valid_rows, z_ref, st_ref, w_ref, z2_ref, stats_ref):
    i = pl.program_id(0)
    m = st_ref[0:1] / cnt
    v = st_ref[1:2] / cnt - m * m
    a = jax.nn.relu((z_ref[...] - m) / jnp.sqrt(v + 1e-5))
    z2 = _bf16_dot(a, w_ref[...])
    z2_ref[...] = z2
    _acc_stats(stats_ref, z2, valid_rows, i == 0)


def _passC_body(cnt, valid_rows, z2_ref, st_ref, x_ref, cs_ref):
    i = pl.program_id(0)
    m = st_ref[0:1] / cnt
    v = st_ref[1:2] / cnt - m * m
    xn = jax.nn.relu((z2_ref[...] - m) / jnp.sqrt(v + 1e-5))
    x_ref[...] = xn

    @pl.when(i == 0)
    def _():
        cs_ref[...] = jnp.zeros_like(cs_ref)

    if valid_rows is not None:
        mask = lax.broadcasted_iota(jnp.int32, xn.shape, 0) < valid_rows
        xn = jnp.where(mask, xn, 0.0)
    cs_ref[0:1] += jnp.sum(xn, axis=0, keepdims=True)


def _mlp_layer(x, agg, w1, w2, nrows, blk, valid_rows, cnt):
    grid = (nrows // blk,)
    d1 = w1.shape[1]
    z1, st1 = pl.pallas_call(
        functools.partial(_passA_body, valid_rows),
        grid=grid,
        in_specs=[
            pl.BlockSpec((blk, H), lambda i: (i, 0)),
            pl.BlockSpec((1, blk, H), lambda i: (0, i, 0)),
            pl.BlockSpec((1, blk, H), lambda i: (1, i, 0)),
            pl.BlockSpec((H, d1), lambda i: (0, 0)),
        ],
        out_specs=[
            pl.BlockSpec((blk, d1), lambda i: (i, 0)),
            pl.BlockSpec((8, d1), lambda i: (0, 0)),
        ],
        out_shape=[jax.ShapeDtypeStruct((nrows, d1), jnp.float32),
                   jax.ShapeDtypeStruct((8, d1), jnp.float32)],
    )(x, agg, agg, w1)
    z2, st2 = pl.pallas_call(
        functools.partial(_passB_body, cnt, valid_rows),
        grid=grid,
        in_specs=[
            pl.BlockSpec((blk, d1), lambda i: (i, 0)),
            pl.BlockSpec((8, d1), lambda i: (0, 0)),
            pl.BlockSpec((d1, H), lambda i: (0, 0)),
        ],
        out_specs=[
            pl.BlockSpec((blk, H), lambda i: (i, 0)),
            pl.BlockSpec((8, H), lambda i: (0, 0)),
        ],
        out_shape=[jax.ShapeDtypeStruct((nrows, H), jnp.float32),
                   jax.ShapeDtypeStruct((8, H), jnp.float32)],
    )(z1, st1, w2)
    xn, cs = pl.pallas_call(
        functools.partial(_passC_body, cnt, valid_rows),
        grid=grid,
        in_specs=[
            pl.BlockSpec((blk, H), lambda i: (i, 0)),
            pl.BlockSpec((8, H), lambda i: (0, 0)),
        ],
        out_specs=[
            pl.BlockSpec((blk, H), lambda i: (i, 0)),
            pl.BlockSpec((8, H), lambda i: (0, 0)),
        ],
        out_shape=[jax.ShapeDtypeStruct((nrows, H), jnp.float32),
                   jax.ShapeDtypeStruct((8, H), jnp.float32)],
    )(z2, st2)
    return xn, cs


def _readout_body(csx_ref, csr_ref, wa_ref, wr_ref, wl_ref, o_ref):
    xm = csx_ref[0:1] / jnp.float32(N)
    rgm = csr_ref[0:1] / jnp.float32(NRG)
    xp = _bf16_dot(xm, wa_ref[...])
    rgp = _bf16_dot(rgm, wr_ref[...])
    pre = jax.nn.relu(xp + rgp)
    o_ref[...] = jnp.sum(pre * wl_ref[...].T, axis=1, keepdims=True)


def kernel(node_feat, edge_index, edge_feat, rg_atom_features_0,
           rg_edge_index_0, mapping_0, atom_emb, rg_emb, bond_emb,
           atom_w1, atom_w2, rg_w1, rg_w2, atom_lin_w, rg_lin_w, lin_w):
    i32 = jnp.int32

    atab = atom_emb.reshape(900, H)
    aidx = (node_feat.astype(i32) + 100 * jnp.arange(9, dtype=i32)[None, :]).T
    aidx = jnp.pad(aidx, ((0, 0), (0, NPAD_ENC - N))).reshape(9, NW, ENC_CH, 128)
    ridx = jnp.pad(rg_atom_features_0.astype(i32),
                   (0, NRG_PAD - NRG)).reshape(NW, 112)

    src = edge_index[0].astype(i32)
    dst = edge_index[1].astype(i32)
    srcp = jnp.pad(src, (0, EPAD - E)).reshape(NW, ECH, 128)
    dstp = jnp.concatenate(
        [dst, N + (jnp.arange(EPAD - E, dtype=i32) % (NAGG - N))]
    ).reshape(NW, ECH, 128)
    ef = edge_feat.astype(i32)
    c01p = jnp.pad(ef[:, 0] * 100 + ef[:, 1],
                   (0, EPAD - E)).reshape(NW, ECH, 128)
    c2p = jnp.pad(ef[:, 2], (0, EPAD - E)).reshape(NW, ECH, 128)

    rsrc = rg_edge_index_0[0].astype(i32)
    rdst = rg_edge_index_0[1].astype(i32)
    rsrcp = jnp.pad(rsrc, (0, REPAD - ERG)).reshape(NW, RECH, 128)
    rdstp = jnp.concatenate(
        [rdst, NRG + (jnp.arange(REPAD - ERG, dtype=i32) % (NRG_PAD - NRG))]
    ).reshape(NW, RECH, 128)

    x, rg = _sc_encode(atab, rg_emb, aidx, ridx)

    csx = csr = None
    for i in range(L):
        t01 = _pair_table(bond_emb[i, 0], bond_emb[i, 1])
        agg = _sc_edge(x, t01, bond_emb[i, 2], srcp, c01p, c2p, dstp)
        ragg = _sc_rg(rg, rsrcp, rdstp)
        x, csx = _mlp_layer(x, agg, atom_w1[i], atom_w2[i],
                            N, 2000, None, float(N))
        rg, csr = _mlp_layer(rg, ragg, rg_w1[i], rg_w2[i],
                             NRG_PAD, NRG_PAD, NRG, float(NRG))

    return pl.pallas_call(
        _readout_body,
        out_shape=jax.ShapeDtypeStruct((1, 1), jnp.float32),
    )(csx, csr, atom_lin_w, rg_lin_w, lin_w)


# speedup vs baseline: 1.8842x; 1.8842x over previous
import functools

import jax
import jax.numpy as jnp
from jax import lax
from jax.experimental import pallas as pl
from jax.experimental.pallas import tpu as pltpu
from jax.experimental.pallas import tpu_sc as plsc

N = 10000
E = 320000
H = 128
NRG = 3333
ERG = 6666
L = 2

NC = 2
NS = 16
NW = NC * NS

ENC_CH = 3
NPAD_ENC = NW * ENC_CH * 128
NRG_PAD = 3584
RG_PER_SUB = NRG_PAD // NS
ECH = 80
EGRP = 8
EPW = ECH * 128
EPAD = NW * EPW
NAGG = 10112
AGG_PER_SUB = NAGG // NS
RECH = 2
REPW = RECH * 128
REPAD = NW * REPW

_mesh = plsc.VectorSubcoreMesh(core_axis_name="c", subcore_axis_name="s")


def _wid():
    return lax.axis_index("c") * NS + lax.axis_index("s")


def _zero_buf(buf, rows):
    z = jnp.zeros((16,), jnp.float32)

    @pl.loop(0, rows)
    def _(r):
        for k in range(H // 16):
            buf[r, pl.ds(16 * k, 16)] = z


def _relu_buf(buf, rows):
    @pl.loop(0, rows)
    def _(r):
        for k in range(H // 16):
            v = buf[r, pl.ds(16 * k, 16)]
            buf[r, pl.ds(16 * k, 16)] = jnp.maximum(v, 0.0)


def _encode_body(atab_hbm, rgtab_hbm, aidx_hbm, ridx_hbm,
                 x0_hbm, rg0_hbm,
                 atab_sh, rgtab_sh, aidx_v, ridx_v, buf, sem):
    cid = lax.axis_index("c")
    sid = lax.axis_index("s")
    w = cid * NS + sid

    @pl.when(sid == 0)
    def _():
        pltpu.sync_copy(atab_hbm, atab_sh)
        pltpu.sync_copy(rgtab_hbm, rgtab_sh)

    plsc.subcore_barrier()

    for f in range(9):
        pltpu.sync_copy(aidx_hbm.at[f, w], aidx_v.at[f])
    pltpu.sync_copy(ridx_hbm.at[w], ridx_v)

    @pl.loop(0, ENC_CH)
    def _(c):
        pltpu.async_copy(atab_sh.at[aidx_v.at[0, c]], buf, sem).wait()
        for f in range(1, 9):
            pltpu.async_copy(atab_sh.at[aidx_v.at[f, c]], buf, sem,
                             add=True).wait()
        pltpu.sync_copy(buf, x0_hbm.at[pl.ds((w * ENC_CH + c) * 128, 128)])

    rbuf = buf.at[pl.ds(0, 112)]
    pltpu.async_copy(rgtab_sh.at[ridx_v], rbuf, sem).wait()
    pltpu.sync_copy(rbuf, rg0_hbm.at[pl.ds(w * 112, 112)])


def _sc_encode(atab, rgtab, aidx, ridx):
    k = pl.kernel(
        _encode_body,
        out_type=[jax.ShapeDtypeStruct((NPAD_ENC, H), jnp.float32),
                  jax.ShapeDtypeStruct((NRG_PAD, H), jnp.float32)],
        mesh=_mesh,
        scratch_types=[
            pltpu.VMEM_SHARED((900, H), jnp.float32),
            pltpu.VMEM_SHARED((8, H), jnp.float32),
            pltpu.VMEM((9, ENC_CH, 128), jnp.int32),
            pltpu.VMEM((112,), jnp.int32),
            pltpu.VMEM((128, H), jnp.float32),
            pltpu.SemaphoreType.DMA,
        ],
    )
    return k(atab, rgtab, aidx, ridx)


def _edge_body(x_hbm, bond_hbm, src_hbm, cb_hbm, dst_hbm,
               agg_hbm,
               bond_sh, agg_sh,
               srcv, cbv, dstv, buf0, buf1, sem0, sem1, isem):
    cid = lax.axis_index("c")
    sid = lax.axis_index("s")
    w = cid * NS + sid

    @pl.when(sid == 0)
    def _():
        pltpu.sync_copy(bond_hbm, bond_sh)

    _zero_buf(buf0, 128)
    for k in range(AGG_PER_SUB // 128):
        pltpu.sync_copy(buf0, agg_sh.at[pl.ds(sid * AGG_PER_SUB + 128 * k, 128)])
    rem = AGG_PER_SUB % 128
    if rem:
        pltpu.sync_copy(
            buf0.at[pl.ds(0, rem)],
            agg_sh.at[pl.ds(sid * AGG_PER_SUB + AGG_PER_SUB - rem, rem)])
    plsc.subcore_barrier()

    bufs = (buf0, buf1)
    sems = (sem0, sem1)

    @pl.loop(0, ECH // EGRP)
    def _(g):
        ds = [pltpu.async_copy(src_hbm.at[w, pl.ds(g * EGRP, EGRP)], srcv, isem),
              pltpu.async_copy(dst_hbm.at[w, pl.ds(g * EGRP, EGRP)], dstv, isem)]
        for j in range(3):
            ds.append(pltpu.async_copy(cb_hbm.at[j, w, pl.ds(g * EGRP, EGRP)],
                                       cbv.at[j], isem))
        for d in ds:
            d.wait()
        pltpu.async_copy(x_hbm.at[srcv.at[0]], bufs[0], sems[0]).wait()
        for c in range(EGRP):
            cur = c % 2
            nxt = 1 - cur
            adds = [pltpu.async_copy(bond_sh.at[cbv.at[j, c]], bufs[cur],
                                     sems[cur], add=True) for j in range(3)]
            if c + 1 < EGRP:
                xn = pltpu.async_copy(x_hbm.at[srcv.at[c + 1]],
                                      bufs[nxt], sems[nxt])
            for a in adds:
                a.wait()
            _relu_buf(bufs[cur], 128)
            pltpu.sync_copy(bufs[cur], agg_sh.at[dstv.at[c]], add=True)
            if c + 1 < EGRP:
                xn.wait()

    plsc.subcore_barrier()

    pltpu.sync_copy(agg_sh.at[pl.ds(sid * AGG_PER_SUB, AGG_PER_SUB)],
                    agg_hbm.at[cid, pl.ds(sid * AGG_PER_SUB, AGG_PER_SUB)])


def _sc_edge(x, bond, srcp, cbp, dstp):
    k = pl.kernel(
        _edge_body,
        out_type=[jax.ShapeDtypeStruct((NC, NAGG, H), jnp.float32)],
        mesh=_mesh,
        scratch_types=[
            pltpu.VMEM_SHARED((300, H), jnp.float32),
            pltpu.VMEM_SHARED((NAGG, H), jnp.float32),
            pltpu.VMEM((EGRP, 128), jnp.int32),
            pltpu.VMEM((3, EGRP, 128), jnp.int32),
            pltpu.VMEM((EGRP, 128), jnp.int32),
            pltpu.VMEM((128, H), jnp.float32),
            pltpu.VMEM((128, H), jnp.float32),
            pltpu.SemaphoreType.DMA,
            pltpu.SemaphoreType.DMA,
            pltpu.SemaphoreType.DMA,
        ],
    )
    (agg,) = k(x, bond, srcp, cbp, dstp)
    return agg


def _rg_body(rg_hbm, rsrc_hbm, rdst_hbm, ragg_hbm,
             ragg_sh, rsv, rdv, buf, sem):
    cid = lax.axis_index("c")
    sid = lax.axis_index("s")
    w = cid * NS + sid

    _zero_buf(buf, 128)
    pltpu.sync_copy(buf, ragg_sh.at[pl.ds(sid * RG_PER_SUB, 128)])
    pltpu.sync_copy(buf.at[pl.ds(0, RG_PER_SUB - 128)],
                    ragg_sh.at[pl.ds(sid * RG_PER_SUB + 128, RG_PER_SUB - 128)])
    plsc.subcore_barrier()

    pltpu.sync_copy(rsrc_hbm.at[w], rsv)
    pltpu.sync_copy(rdst_hbm.at[w], rdv)

    for c in range(RECH):
        pltpu.async_copy(rg_hbm.at[rsv.at[c]], buf, sem).wait()
        pltpu.sync_copy(buf, ragg_sh.at[rdv.at[c]], add=True)

    plsc.subcore_barrier()
    pltpu.sync_copy(ragg_sh.at[pl.ds(sid * RG_PER_SUB, RG_PER_SUB)],
                    ragg_hbm.at[cid, pl.ds(sid * RG_PER_SUB, RG_PER_SUB)])


def _sc_rg(rg, rsrcp, rdstp):
    k = pl.kernel(
        _rg_body,
        out_type=[jax.ShapeDtypeStruct((NC, NRG_PAD, H), jnp.float32)],
        mesh=_mesh,
        scratch_types=[
            pltpu.VMEM_SHARED((NRG_PAD, H), jnp.float32),
            pltpu.VMEM((RECH, 128), jnp.int32),
            pltpu.VMEM((RECH, 128), jnp.int32),
            pltpu.VMEM((128, H), jnp.float32),
            pltpu.SemaphoreType.DMA,
        ],
    )
    (ragg,) = k(rg, rsrcp, rdstp)
    return ragg


def _bf16_dot(a, b):
    return jnp.dot(a.astype(jnp.bfloat16), b.astype(jnp.bfloat16),
                   preferred_element_type=jnp.float32)


def _acc_stats(stats_ref, z, valid_rows, first):
    @pl.when(first)
    def _():
        stats_ref[...] = jnp.zeros_like(stats_ref)

    if valid_rows is not None:
        mask = lax.broadcasted_iota(jnp.int32, z.shape, 0) < valid_rows
        z = jnp.where(mask, z, 0.0)
    stats_ref[0:1] += jnp.sum(z, axis=0, keepdims=True)
    stats_ref[1:2] += jnp.sum(z * z, axis=0, keepdims=True)


def _passA_body(valid_rows, x_ref, a0_ref, a1_ref, w_ref, z_ref, stats_ref):
    i = pl.program_id(0)
    h = x_ref[...] + (a0_ref[0] + a1_ref[0])
    z = _bf16_dot(h, w_ref[...])
    z_ref[...] = z
    _acc_stats(stats_ref, z, valid_rows, i == 0)


def _passB_body(cnt, valid_rows, z_ref, st_ref, w_ref, z2_ref, stats_ref):
    i = pl.program_id(0)
    m = st_ref[0:1] / cnt
    v = st_ref[1:2] / cnt - m * m
    a = jax.nn.relu((z_ref[...] - m) / jnp.sqrt(v + 1e-5))
    z2 = _bf16_dot(a, w_ref[...])
    z2_ref[...] = z2
    _acc_stats(stats_ref, z2, valid_rows, i == 0)


def _passC_body(cnt, valid_rows, z2_ref, st_ref, x_ref, cs_ref):
    i = pl.program_id(0)
    m = st_ref[0:1] / cnt
    v = st_ref[1:2] / cnt - m * m
    xn = jax.nn.relu((z2_ref[...] - m) / jnp.sqrt(v + 1e-5))
    x_ref[...] = xn

    @pl.when(i == 0)
    def _():
        cs_ref[...] = jnp.zeros_like(cs_ref)

    if valid_rows is not None:
        mask = lax.broadcasted_iota(jnp.int32, xn.shape, 0) < valid_rows
        xn = jnp.where(mask, xn, 0.0)
    cs_ref[0:1] += jnp.sum(xn, axis=0, keepdims=True)


def _mlp_layer(x, agg, w1, w2, nrows, blk, valid_rows, cnt):
    grid = (nrows // blk,)
    d1 = w1.shape[1]
    z1, st1 = pl.pallas_call(
        functools.partial(_passA_body, valid_rows),
        grid=grid,
        in_specs=[
            pl.BlockSpec((blk, H), lambda i: (i, 0)),
            pl.BlockSpec((1, blk, H), lambda i: (0, i, 0)),
            pl.BlockSpec((1, blk, H), lambda i: (1, i, 0)),
            pl.BlockSpec((H, d1), lambda i: (0, 0)),
        ],
        out_specs=[
            pl.BlockSpec((blk, d1), lambda i: (i, 0)),
            pl.BlockSpec((8, d1), lambda i: (0, 0)),
        ],
        out_shape=[jax.ShapeDtypeStruct((nrows, d1), jnp.float32),
                   jax.ShapeDtypeStruct((8, d1), jnp.float32)],
    )(x, agg, agg, w1)
    z2, st2 = pl.pallas_call(
        functools.partial(_passB_body, cnt, valid_rows),
        grid=grid,
        in_specs=[
            pl.BlockSpec((blk, d1), lambda i: (i, 0)),
            pl.BlockSpec((8, d1), lambda i: (0, 0)),
            pl.BlockSpec((d1, H), lambda i: (0, 0)),
        ],
        out_specs=[
            pl.BlockSpec((blk, H), lambda i: (i, 0)),
            pl.BlockSpec((8, H), lambda i: (0, 0)),
        ],
        out_shape=[jax.ShapeDtypeStruct((nrows, H), jnp.float32),
                   jax.ShapeDtypeStruct((8, H), jnp.float32)],
    )(z1, st1, w2)
    xn, cs = pl.pallas_call(
        functools.partial(_passC_body, cnt, valid_rows),
        grid=grid,
        in_specs=[
            pl.BlockSpec((blk, H), lambda i: (i, 0)),
            pl.BlockSpec((8, H), lambda i: (0, 0)),
        ],
        out_specs=[
            pl.BlockSpec((blk, H), lambda i: (i, 0)),
            pl.BlockSpec((8, H), lambda i: (0, 0)),
        ],
        out_shape=[jax.ShapeDtypeStruct((nrows, H), jnp.float32),
                   jax.ShapeDtypeStruct((8, H), jnp.float32)],
    )(z2, st2)
    return xn, cs


def _readout_body(csx_ref, csr_ref, wa_ref, wr_ref, wl_ref, o_ref):
    xm = csx_ref[0:1] / jnp.float32(N)
    rgm = csr_ref[0:1] / jnp.float32(NRG)
    xp = _bf16_dot(xm, wa_ref[...])
    rgp = _bf16_dot(rgm, wr_ref[...])
    pre = jax.nn.relu(xp + rgp)
    o_ref[...] = jnp.sum(pre * wl_ref[...].T, axis=1, keepdims=True)


def kernel(node_feat, edge_index, edge_feat, rg_atom_features_0,
           rg_edge_index_0, mapping_0, atom_emb, rg_emb, bond_emb,
           atom_w1, atom_w2, rg_w1, rg_w2, atom_lin_w, rg_lin_w, lin_w):
    i32 = jnp.int32

    atab = atom_emb.reshape(900, H)
    aidx = (node_feat.astype(i32) + 100 * jnp.arange(9, dtype=i32)[None, :]).T
    aidx = jnp.pad(aidx, ((0, 0), (0, NPAD_ENC - N))).reshape(9, NW, ENC_CH, 128)
    ridx = jnp.pad(rg_atom_features_0.astype(i32),
                   (0, NRG_PAD - NRG)).reshape(NW, 112)

    src = edge_index[0].astype(i32)
    dst = edge_index[1].astype(i32)
    srcp = jnp.pad(src, (0, EPAD - E)).reshape(NW, ECH, 128)
    dstp = jnp.concatenate(
        [dst, N + (jnp.arange(EPAD - E, dtype=i32) % (NAGG - N))]
    ).reshape(NW, ECH, 128)
    cb = edge_feat.astype(i32).T + 100 * jnp.arange(3, dtype=i32)[:, None]
    cbp = jnp.pad(cb, ((0, 0), (0, EPAD - E))).reshape(3, NW, ECH, 128)

    rsrc = rg_edge_index_0[0].astype(i32)
    rdst = rg_edge_index_0[1].astype(i32)
    rsrcp = jnp.pad(rsrc, (0, REPAD - ERG)).reshape(NW, RECH, 128)
    rdstp = jnp.concatenate(
        [rdst, NRG + (jnp.arange(REPAD - ERG, dtype=i32) % (NRG_PAD - NRG))]
    ).reshape(NW, RECH, 128)

    x, rg = _sc_encode(atab, rg_emb, aidx, ridx)

    bond = bond_emb.reshape(L, 300, H)
    csx = csr = None
    for i in range(L):
        agg = _sc_edge(x, bond[i], srcp, cbp, dstp)
        ragg = _sc_rg(rg, rsrcp, rdstp)
        x, csx = _mlp_layer(x, agg, atom_w1[i], atom_w2[i],
                            N, 2000, None, float(N))
        rg, csr = _mlp_layer(rg, ragg, rg_w1[i], rg_w2[i],
                             NRG_PAD, NRG_PAD, NRG, float(NRG))

    return pl.pallas_call(
        _readout_body,
        out_shape=jax.ShapeDtypeStruct((1, 1), jnp.float32),
    )(csx, csr, atom_lin_w, rg_lin_w, lin_w)
